# Initial kernel scaffold; baseline (speedup 1.0000x reference)
import jax
import jax.numpy as jnp
from jax.experimental import pallas as pl


def kernel(x, edge_index, W1l, b1l, W1r, W2l, b2l, W2r):
    raise NotImplementedError("write your pallas kernel here")


# speedup vs baseline: 4.2974x; 4.2974x over previous
import functools

import jax
import jax.numpy as jnp
from jax import lax
from jax.experimental import pallas as pl
from jax.experimental.pallas import tpu as pltpu
from jax.experimental.pallas import tpu_sc as plsc

NC = 2
NS = 16
NW = NC * NS
CH = 128


def _make_agg(de, n_pad, cpw):
  mesh = plsc.VectorSubcoreMesh(core_axis_name="c", subcore_axis_name="s",
                                num_cores=NC, num_subcores=NS)
  rows_per_tile = n_pad // NS
  n0 = rows_per_tile // CH

  @functools.partial(
      pl.kernel,
      out_type=jax.ShapeDtypeStruct((NC, n_pad, de), jnp.float32),
      mesh=mesh,
      scratch_types=[
          pltpu.VMEM((CH,), jnp.int32),
          pltpu.VMEM((CH,), jnp.int32),
          pltpu.VMEM((CH, de), jnp.float32),
          pltpu.VMEM((CH, de), jnp.float32),
          pltpu.VMEM_SHARED((n_pad, de), jnp.float32),
          pltpu.SemaphoreType.DMA,
      ],
      compiler_params=pltpu.CompilerParams(use_tc_tiling_on_sc=False),
  )
  def agg(x_hbm, src_hbm, dst_hbm, z_hbm, out_hbm,
          sidx, didx, rows, stage, acc, sem):
    c = lax.axis_index("c")
    s = lax.axis_index("s")
    wid = c * NS + s

    pltpu.sync_copy(z_hbm, stage)
    for k in range(n0):
      pltpu.sync_copy(stage, acc.at[pl.ds(s * rows_per_tile + k * CH, CH)])
    plsc.subcore_barrier()

    base = wid * cpw * CH

    def body(g, carry):
      off = base + g * CH
      pltpu.sync_copy(src_hbm.at[pl.ds(off, CH)], sidx)
      pltpu.sync_copy(dst_hbm.at[pl.ds(off, CH)], didx)
      pltpu.async_copy(x_hbm.at[sidx], rows, sem).wait()
      pltpu.sync_copy(rows, acc.at[didx], add=True)
      return carry

    lax.fori_loop(0, cpw, body, 0)
    plsc.subcore_barrier()

    for k in range(n0):
      r0 = s * rows_per_tile + k * CH
      pltpu.sync_copy(acc.at[pl.ds(r0, CH)], stage)
      pltpu.sync_copy(stage, out_hbm.at[c, pl.ds(r0, CH)])

  return agg


def _tc_layer(a0, a1, c0, c1, x, wlT, bl, wrT, final):
  n, d = x.shape
  bn = 1000 if n % 1000 == 0 else 8
  grid = (n // bn,)

  def body(a0_r, a1_r, c0_r, c1_r, x_r, wl_r, bl_r, wr_r, o_r):
    cnt = c0_r[...] + c1_r[...]
    rec = 1.0 / jnp.maximum(cnt, 1.0)
    mean = (a0_r[...] + a1_r[...]) * rec
    h = jnp.dot(mean, wl_r[...], preferred_element_type=jnp.float32)
    h = h + bl_r[...]
    h = h + jnp.dot(x_r[...], wr_r[...], preferred_element_type=jnp.float32)
    if final:
      m = jnp.max(h, axis=1, keepdims=True)
      h = h - m
      lse = jnp.log(jnp.sum(jnp.exp(h), axis=1, keepdims=True))
      o_r[...] = h - lse
    else:
      o_r[...] = jnp.maximum(h, 0.0)

  row_spec = pl.BlockSpec((bn, d), lambda i: (i, 0))
  cnt_spec = pl.BlockSpec((bn, 1), lambda i: (i, 0))
  w_spec = pl.BlockSpec((d, d), lambda i: (0, 0))
  b_spec = pl.BlockSpec((1, d), lambda i: (0, 0))
  return pl.pallas_call(
      body,
      grid=grid,
      in_specs=[row_spec, row_spec, cnt_spec, cnt_spec, row_spec,
                w_spec, b_spec, w_spec],
      out_specs=row_spec,
      out_shape=jax.ShapeDtypeStruct((n, d), jnp.float32),
  )(a0, a1, c0, c1, x, wlT, bl, wrT)


def kernel(x, edge_index, W1l, b1l, W1r, W2l, b2l, W2r):
  n, d = x.shape
  e = edge_index.shape[1]
  de = d + 16
  n_pad = -(-(n + 1) // (NS * CH)) * (NS * CH)
  cpw = -(-e // (NW * CH))
  e_pad = NW * CH * cpw

  src = edge_index[0]
  dst = edge_index[1]
  pad = e_pad - e
  if pad:
    src = jnp.concatenate([src, jnp.zeros((pad,), jnp.int32)])
    dst = jnp.concatenate([dst, jnp.full((pad,), n, jnp.int32)])

  x_ext = jnp.concatenate(
      [x, jnp.ones((n, 1), jnp.float32), jnp.zeros((n, 15), jnp.float32)],
      axis=1)

  agg1 = _make_agg(de, n_pad, cpw)
  p = agg1(x_ext, src, dst, jnp.zeros((CH, de), jnp.float32))
  a0, a1 = p[0, :n, :d], p[1, :n, :d]
  c0, c1 = p[0, :n, d:d + 1], p[1, :n, d:d + 1]

  h1 = _tc_layer(a0, a1, c0, c1, x, W1l.T, b1l.reshape(1, d), W1r.T,
                 final=False)

  agg2 = _make_agg(d, n_pad, cpw)
  q = agg2(h1, src, dst, jnp.zeros((CH, d), jnp.float32))

  return _tc_layer(q[0, :n], q[1, :n], c0, c1, h1, W2l.T,
                   b2l.reshape(1, d), W2r.T, final=True)


# speedup vs baseline: 5.4303x; 1.2636x over previous
import functools

import jax
import jax.numpy as jnp
from jax import lax
from jax.experimental import pallas as pl
from jax.experimental.pallas import tpu as pltpu
from jax.experimental.pallas import tpu_sc as plsc

NC = 2
NS = 16
NW = NC * NS
CH = 128


def _make_agg(de, n_pad, cpw):
  mesh = plsc.VectorSubcoreMesh(core_axis_name="c", subcore_axis_name="s",
                                num_cores=NC, num_subcores=NS)
  rows_per_tile = n_pad // NS
  n0 = rows_per_tile // CH

  @functools.partial(
      pl.kernel,
      out_type=jax.ShapeDtypeStruct((NC, n_pad, de), jnp.float32),
      mesh=mesh,
      scratch_types=[
          pltpu.VMEM((2, CH), jnp.int32),
          pltpu.VMEM((2, CH), jnp.int32),
          pltpu.VMEM((CH, de), jnp.float32),
          pltpu.VMEM((CH, de), jnp.float32),
          pltpu.VMEM_SHARED((n_pad, de), jnp.float32),
          pltpu.SemaphoreType.DMA,
          pltpu.SemaphoreType.DMA,
      ],
      compiler_params=pltpu.CompilerParams(use_tc_tiling_on_sc=False),
  )
  def agg(x_hbm, src_hbm, dst_hbm, z_hbm, out_hbm,
          sidx, didx, rows0, rows1, acc, sem0, sem1):
    c = lax.axis_index("c")
    s = lax.axis_index("s")
    wid = c * NS + s
    rows = (rows0, rows1)
    sems = (sem0, sem1)

    pltpu.sync_copy(z_hbm, rows0)
    for k in range(n0):
      pltpu.sync_copy(rows0, acc.at[pl.ds(s * rows_per_tile + k * CH, CH)])
    plsc.subcore_barrier()

    base = wid * cpw * CH

    def load_idx(g, b):
      off = base + g * CH
      pltpu.sync_copy(src_hbm.at[pl.ds(off, CH)], sidx.at[b])
      pltpu.sync_copy(dst_hbm.at[pl.ds(off, CH)], didx.at[b])

    def start_gather(b):
      pltpu.async_copy(x_hbm.at[sidx.at[b]], rows[b], sems[b])

    def wait_gather(b):
      pltpu.make_async_copy(x_hbm.at[sidx.at[b]], rows[b], sems[b]).wait()

    def scatter(b):
      pltpu.sync_copy(rows[b], acc.at[didx.at[b]], add=True)

    load_idx(0, 0)
    start_gather(0)

    @pl.loop(0, cpw - 1, step=2)
    def _pipe(g):
      for b in range(2):
        load_idx(g + b + 1, 1 - b)
        wait_gather(b)
        start_gather(1 - b)
        scatter(b)

    wait_gather((cpw - 1) % 2)
    scatter((cpw - 1) % 2)
    plsc.subcore_barrier()

    for k in range(n0):
      r0 = s * rows_per_tile + k * CH
      b = k % 2
      pltpu.sync_copy(acc.at[pl.ds(r0, CH)], rows[b])
      pltpu.sync_copy(rows[b], out_hbm.at[c, pl.ds(r0, CH)])

  return agg


def _tc_layer(a0, a1, c0, c1, x, wlT, bl, wrT, final):
  n, d = x.shape
  bn = 1000 if n % 1000 == 0 else 8
  grid = (n // bn,)

  def body(a0_r, a1_r, c0_r, c1_r, x_r, wl_r, bl_r, wr_r, o_r):
    cnt = c0_r[...] + c1_r[...]
    rec = 1.0 / jnp.maximum(cnt, 1.0)
    mean = (a0_r[...] + a1_r[...]) * rec
    h = jnp.dot(mean, wl_r[...], preferred_element_type=jnp.float32)
    h = h + bl_r[...]
    h = h + jnp.dot(x_r[...], wr_r[...], preferred_element_type=jnp.float32)
    if final:
      m = jnp.max(h, axis=1, keepdims=True)
      h = h - m
      lse = jnp.log(jnp.sum(jnp.exp(h), axis=1, keepdims=True))
      o_r[...] = h - lse
    else:
      o_r[...] = jnp.maximum(h, 0.0)

  row_spec = pl.BlockSpec((bn, d), lambda i: (i, 0))
  cnt_spec = pl.BlockSpec((bn, 1), lambda i: (i, 0))
  w_spec = pl.BlockSpec((d, d), lambda i: (0, 0))
  b_spec = pl.BlockSpec((1, d), lambda i: (0, 0))
  return pl.pallas_call(
      body,
      grid=grid,
      in_specs=[row_spec, row_spec, cnt_spec, cnt_spec, row_spec,
                w_spec, b_spec, w_spec],
      out_specs=row_spec,
      out_shape=jax.ShapeDtypeStruct((n, d), jnp.float32),
  )(a0, a1, c0, c1, x, wlT, bl, wrT)


def kernel(x, edge_index, W1l, b1l, W1r, W2l, b2l, W2r):
  n, d = x.shape
  e = edge_index.shape[1]
  de = d + 16
  n_pad = -(-(n + 1) // (NS * CH)) * (NS * CH)
  cpw = -(-e // (NW * CH))
  if cpw % 2 == 0:
    cpw += 1
  e_pad = NW * CH * cpw

  src = edge_index[0]
  dst = edge_index[1]
  pad = e_pad - e
  if pad:
    src = jnp.concatenate([src, jnp.zeros((pad,), jnp.int32)])
    dst = jnp.concatenate([dst, jnp.full((pad,), n, jnp.int32)])

  x_ext = jnp.concatenate(
      [x, jnp.ones((n, 1), jnp.float32), jnp.zeros((n, 15), jnp.float32)],
      axis=1)

  agg1 = _make_agg(de, n_pad, cpw)
  p = agg1(x_ext, src, dst, jnp.zeros((CH, de), jnp.float32))
  a0, a1 = p[0, :n, :d], p[1, :n, :d]
  c0, c1 = p[0, :n, d:d + 1], p[1, :n, d:d + 1]

  h1 = _tc_layer(a0, a1, c0, c1, x, W1l.T, b1l.reshape(1, d), W1r.T,
                 final=False)

  agg2 = _make_agg(d, n_pad, cpw)
  q = agg2(h1, src, dst, jnp.zeros((CH, d), jnp.float32))

  return _tc_layer(q[0, :n], q[1, :n], c0, c1, h1, W2l.T,
                   b2l.reshape(1, d), W2r.T, final=True)


# speedup vs baseline: 5.7688x; 1.0623x over previous
import functools

import jax
import jax.numpy as jnp
from jax import lax
from jax.experimental import pallas as pl
from jax.experimental.pallas import tpu as pltpu
from jax.experimental.pallas import tpu_sc as plsc

NC = 2
NS = 16
NW = NC * NS
CH = 128


def _make_agg(de, n_pad, cpw):
  mesh = plsc.VectorSubcoreMesh(core_axis_name="c", subcore_axis_name="s",
                                num_cores=NC, num_subcores=NS)
  rows_per_tile = n_pad // NS
  n0 = rows_per_tile // CH

  @functools.partial(
      pl.kernel,
      out_type=jax.ShapeDtypeStruct((NC, n_pad, de), jnp.float32),
      mesh=mesh,
      scratch_types=[
          pltpu.VMEM((2, CH), jnp.int32),
          pltpu.VMEM((2, CH), jnp.int32),
          pltpu.VMEM((CH, de), jnp.float32),
          pltpu.VMEM((CH, de), jnp.float32),
          pltpu.VMEM_SHARED((n_pad, de), jnp.float32),
          pltpu.SemaphoreType.DMA,
          pltpu.SemaphoreType.DMA,
          pltpu.SemaphoreType.DMA,
          pltpu.SemaphoreType.DMA,
      ],
      compiler_params=pltpu.CompilerParams(use_tc_tiling_on_sc=False),
  )
  def agg(x_hbm, src_hbm, dst_hbm, z_hbm, out_hbm,
          sidx, didx, rows0, rows1, acc, gsem0, gsem1, ssem0, ssem1):
    c = lax.axis_index("c")
    s = lax.axis_index("s")
    wid = c * NS + s
    rows = (rows0, rows1)
    gsems = (gsem0, gsem1)
    ssems = (ssem0, ssem1)

    pltpu.sync_copy(z_hbm, rows0)
    for k in range(n0):
      pltpu.sync_copy(rows0, acc.at[pl.ds(s * rows_per_tile + k * CH, CH)])
    plsc.subcore_barrier()

    base = wid * cpw * CH

    def load_idx(g, b):
      off = base + g * CH
      pltpu.sync_copy(src_hbm.at[pl.ds(off, CH)], sidx.at[b])
      pltpu.sync_copy(dst_hbm.at[pl.ds(off, CH)], didx.at[b])

    def start_gather(b):
      pltpu.async_copy(x_hbm.at[sidx.at[b]], rows[b], gsems[b])

    def wait_gather(b):
      pltpu.make_async_copy(x_hbm.at[sidx.at[b]], rows[b], gsems[b]).wait()

    def start_scatter(b):
      pltpu.async_copy(rows[b], acc.at[didx.at[b]], ssems[b], add=True)

    def wait_scatter(b):
      pltpu.make_async_copy(rows[b], acc.at[didx.at[b]], ssems[b]).wait()

    load_idx(0, 0)
    start_gather(0)
    load_idx(1, 1)
    wait_gather(0)
    start_gather(1)
    start_scatter(0)

    wait_scatter(0)
    load_idx(2, 0)
    wait_gather(1)
    start_gather(0)
    start_scatter(1)

    @pl.loop(2, cpw - 1, step=2)
    def _pipe(g):
      for b in range(2):
        wait_scatter(1 - b)
        load_idx(g + b + 1, 1 - b)
        wait_gather(b)
        start_gather(1 - b)
        start_scatter(b)

    wait_scatter(1)
    wait_gather(0)
    start_scatter(0)
    wait_scatter(0)
    plsc.subcore_barrier()

    for k in range(n0):
      r0 = s * rows_per_tile + k * CH
      b = k % 2
      pltpu.sync_copy(acc.at[pl.ds(r0, CH)], rows[b])
      pltpu.sync_copy(rows[b], out_hbm.at[c, pl.ds(r0, CH)])

  return agg


def _tc_layer1(p, x, wlT, bl, wrT):
  n, d = x.shape
  dp = p.shape[2]
  bn = 1000 if n % 1000 == 0 else 8
  grid = (n // bn,)

  def body(p_r, x_r, wl_r, bl_r, wr_r, h_r, rec_r):
    a = p_r[0] + p_r[1]
    rec = 1.0 / jnp.maximum(a[:, d:d + 1], 1.0)
    mean = a[:, :d] * rec
    h = jnp.dot(mean, wl_r[...], preferred_element_type=jnp.float32)
    h = h + bl_r[...]
    h = h + jnp.dot(x_r[...], wr_r[...], preferred_element_type=jnp.float32)
    h_r[...] = jnp.maximum(h, 0.0)
    rec_r[...] = rec

  return pl.pallas_call(
      body,
      grid=grid,
      in_specs=[
          pl.BlockSpec((2, bn, dp), lambda i: (0, i, 0)),
          pl.BlockSpec((bn, d), lambda i: (i, 0)),
          pl.BlockSpec((d, d), lambda i: (0, 0)),
          pl.BlockSpec((1, d), lambda i: (0, 0)),
          pl.BlockSpec((d, d), lambda i: (0, 0)),
      ],
      out_specs=[
          pl.BlockSpec((bn, d), lambda i: (i, 0)),
          pl.BlockSpec((bn, 1), lambda i: (i, 0)),
      ],
      out_shape=[
          jax.ShapeDtypeStruct((n, d), jnp.float32),
          jax.ShapeDtypeStruct((n, 1), jnp.float32),
      ],
  )(p, x, wlT, bl, wrT)


def _tc_layer2(q, rec, h1, wlT, bl, wrT):
  n, d = h1.shape
  bn = 1000 if n % 1000 == 0 else 8
  grid = (n // bn,)

  def body(q_r, rec_r, h1_r, wl_r, bl_r, wr_r, o_r):
    mean = (q_r[0] + q_r[1]) * rec_r[...]
    h = jnp.dot(mean, wl_r[...], preferred_element_type=jnp.float32)
    h = h + bl_r[...]
    h = h + jnp.dot(h1_r[...], wr_r[...], preferred_element_type=jnp.float32)
    h = h - jnp.max(h, axis=1, keepdims=True)
    o_r[...] = h - jnp.log(jnp.sum(jnp.exp(h), axis=1, keepdims=True))

  return pl.pallas_call(
      body,
      grid=grid,
      in_specs=[
          pl.BlockSpec((2, bn, d), lambda i: (0, i, 0)),
          pl.BlockSpec((bn, 1), lambda i: (i, 0)),
          pl.BlockSpec((bn, d), lambda i: (i, 0)),
          pl.BlockSpec((d, d), lambda i: (0, 0)),
          pl.BlockSpec((1, d), lambda i: (0, 0)),
          pl.BlockSpec((d, d), lambda i: (0, 0)),
      ],
      out_specs=pl.BlockSpec((bn, d), lambda i: (i, 0)),
      out_shape=jax.ShapeDtypeStruct((n, d), jnp.float32),
  )(q, rec, h1, wlT, bl, wrT)


def kernel(x, edge_index, W1l, b1l, W1r, W2l, b2l, W2r):
  n, d = x.shape
  e = edge_index.shape[1]
  de = d + 16
  n_pad = -(-(n + 1) // (NS * CH)) * (NS * CH)
  cpw = -(-e // (NW * CH))
  if cpw % 2 == 0:
    cpw += 1
  e_pad = NW * CH * cpw

  src = edge_index[0]
  dst = edge_index[1]
  pad = e_pad - e
  if pad:
    src = jnp.concatenate([src, jnp.zeros((pad,), jnp.int32)])
    dst = jnp.concatenate([dst, jnp.full((pad,), n, jnp.int32)])

  x_ext = jnp.concatenate(
      [x, jnp.ones((n, 1), jnp.float32), jnp.zeros((n, 15), jnp.float32)],
      axis=1)

  agg1 = _make_agg(de, n_pad, cpw)
  p = agg1(x_ext, src, dst, jnp.zeros((CH, de), jnp.float32))
  h1, rec = _tc_layer1(p, x, W1l.T, b1l.reshape(1, d), W1r.T)

  agg2 = _make_agg(d, n_pad, cpw)
  q = agg2(h1, src, dst, jnp.zeros((CH, d), jnp.float32))

  return _tc_layer2(q, rec, h1, W2l.T, b2l.reshape(1, d), W2r.T)


# speedup vs baseline: 9.4832x; 1.6439x over previous
import functools

import jax
import jax.numpy as jnp
from jax import lax
from jax.experimental import pallas as pl
from jax.experimental.pallas import tpu as pltpu
from jax.experimental.pallas import tpu_sc as plsc

NC = 2
NS = 16
NW = NC * NS
CH = 128


def _make_agg(de, n_pad, cpw):
  mesh = plsc.VectorSubcoreMesh(core_axis_name="c", subcore_axis_name="s",
                                num_cores=NC, num_subcores=NS)
  rows_per_tile = n_pad // NS
  n0 = rows_per_tile // CH

  @functools.partial(
      pl.kernel,
      out_type=jax.ShapeDtypeStruct((NC, n_pad, de), jnp.float32),
      mesh=mesh,
      scratch_types=[
          pltpu.VMEM((2, CH), jnp.int32),
          pltpu.VMEM((2, CH), jnp.int32),
          pltpu.VMEM((CH, de), jnp.float32),
          pltpu.VMEM((CH, de), jnp.float32),
          pltpu.VMEM_SHARED((n_pad, de), jnp.float32),
          pltpu.SemaphoreType.DMA,
          pltpu.SemaphoreType.DMA,
          pltpu.SemaphoreType.DMA,
          pltpu.SemaphoreType.DMA,
      ],
      compiler_params=pltpu.CompilerParams(use_tc_tiling_on_sc=False),
  )
  def agg(x_hbm, src_hbm, dst_hbm, z_hbm, out_hbm,
          sidx, didx, rows0, rows1, acc, gsem0, gsem1, ssem0, ssem1):
    c = lax.axis_index("c")
    s = lax.axis_index("s")
    wid = c * NS + s
    rows = (rows0, rows1)
    gsems = (gsem0, gsem1)
    ssems = (ssem0, ssem1)

    pltpu.sync_copy(z_hbm, rows0)
    for k in range(n0):
      pltpu.sync_copy(rows0, acc.at[pl.ds(s * rows_per_tile + k * CH, CH)])
    plsc.subcore_barrier()

    base = wid * cpw * CH

    def load_idx(g, b):
      off = base + g * CH
      pltpu.sync_copy(src_hbm.at[pl.ds(off, CH)], sidx.at[b])
      pltpu.sync_copy(dst_hbm.at[pl.ds(off, CH)], didx.at[b])

    def start_gather(b):
      pltpu.async_copy(x_hbm.at[sidx.at[b]], rows[b], gsems[b])

    def wait_gather(b):
      pltpu.make_async_copy(x_hbm.at[sidx.at[b]], rows[b], gsems[b]).wait()

    def start_scatter(b):
      pltpu.async_copy(rows[b], acc.at[didx.at[b]], ssems[b], add=True)

    def wait_scatter(b):
      pltpu.make_async_copy(rows[b], acc.at[didx.at[b]], ssems[b]).wait()

    load_idx(0, 0)
    start_gather(0)
    load_idx(1, 1)
    wait_gather(0)
    start_gather(1)
    start_scatter(0)

    wait_scatter(0)
    load_idx(2, 0)
    wait_gather(1)
    start_gather(0)
    start_scatter(1)

    @pl.loop(2, cpw - 1, step=2)
    def _pipe(g):
      for b in range(2):
        wait_scatter(1 - b)
        load_idx(g + b + 1, 1 - b)
        wait_gather(b)
        start_gather(1 - b)
        start_scatter(b)

    wait_scatter(1)
    wait_gather(0)
    start_scatter(0)
    wait_scatter(0)
    plsc.subcore_barrier()

    for k in range(n0):
      r0 = s * rows_per_tile + k * CH
      b = k % 2
      pltpu.sync_copy(acc.at[pl.ds(r0, CH)], rows[b])
      pltpu.sync_copy(rows[b], out_hbm.at[c, pl.ds(r0, CH)])

  return agg


def _tc_layer1(p, x, wlT, bl, wrT):
  n, d = x.shape
  dp = p.shape[2]
  bn = 1000 if n % 1000 == 0 else 8
  grid = (n // bn,)

  def body(p_r, x_r, wl_r, bl_r, wr_r, h_r, rec_r):
    a = p_r[0] + p_r[1]
    rec = 1.0 / jnp.maximum(a[:, d:d + 1], 1.0)
    mean = a[:, :d] * rec
    h = jnp.dot(mean, wl_r[...], preferred_element_type=jnp.float32)
    h = h + bl_r[...]
    h = h + jnp.dot(x_r[...], wr_r[...], preferred_element_type=jnp.float32)
    h_r[...] = jnp.maximum(h, 0.0)
    rec_r[...] = rec

  return pl.pallas_call(
      body,
      grid=grid,
      in_specs=[
          pl.BlockSpec((2, bn, dp), lambda i: (0, i, 0)),
          pl.BlockSpec((bn, d), lambda i: (i, 0)),
          pl.BlockSpec((d, d), lambda i: (0, 0)),
          pl.BlockSpec((1, d), lambda i: (0, 0)),
          pl.BlockSpec((d, d), lambda i: (0, 0)),
      ],
      out_specs=[
          pl.BlockSpec((bn, d), lambda i: (i, 0)),
          pl.BlockSpec((bn, 1), lambda i: (i, 0)),
      ],
      out_shape=[
          jax.ShapeDtypeStruct((n, d), jnp.float32),
          jax.ShapeDtypeStruct((n, 1), jnp.float32),
      ],
  )(p, x, wlT, bl, wrT)


def _tc_layer2(q, rec, h1, wlT, bl, wrT):
  n, d = h1.shape
  bn = 1000 if n % 1000 == 0 else 8
  grid = (n // bn,)

  def body(q_r, rec_r, h1_r, wl_r, bl_r, wr_r, o_r):
    mean = (q_r[0] + q_r[1]) * rec_r[...]
    h = jnp.dot(mean, wl_r[...], preferred_element_type=jnp.float32)
    h = h + bl_r[...]
    h = h + jnp.dot(h1_r[...], wr_r[...], preferred_element_type=jnp.float32)
    h = h - jnp.max(h, axis=1, keepdims=True)
    o_r[...] = h - jnp.log(jnp.sum(jnp.exp(h), axis=1, keepdims=True))

  return pl.pallas_call(
      body,
      grid=grid,
      in_specs=[
          pl.BlockSpec((2, bn, d), lambda i: (0, i, 0)),
          pl.BlockSpec((bn, 1), lambda i: (i, 0)),
          pl.BlockSpec((bn, d), lambda i: (i, 0)),
          pl.BlockSpec((d, d), lambda i: (0, 0)),
          pl.BlockSpec((1, d), lambda i: (0, 0)),
          pl.BlockSpec((d, d), lambda i: (0, 0)),
      ],
      out_specs=pl.BlockSpec((bn, d), lambda i: (i, 0)),
      out_shape=jax.ShapeDtypeStruct((n, d), jnp.float32),
  )(q, rec, h1, wlT, bl, wrT)


def kernel(x, edge_index, W1l, b1l, W1r, W2l, b2l, W2r):
  n, d = x.shape
  e = edge_index.shape[1]
  de = d + 16
  n_pad = -(-(n + 1) // (NS * CH)) * (NS * CH)
  cpw = -(-e // (NW * CH))
  if cpw % 2 == 0:
    cpw += 1
  e_pad = NW * CH * cpw

  src = edge_index[0]
  dst = edge_index[1]
  pad = e_pad - e
  if pad:
    pad_idx = jnp.arange(pad, dtype=jnp.int32)
    src = jnp.concatenate([src, pad_idx % n])
    dst = jnp.concatenate([dst, n + pad_idx % (n_pad - n)])

  x_ext = jnp.concatenate(
      [x, jnp.ones((n, 1), jnp.float32), jnp.zeros((n, 15), jnp.float32)],
      axis=1)

  agg1 = _make_agg(de, n_pad, cpw)
  p = agg1(x_ext, src, dst, jnp.zeros((CH, de), jnp.float32))
  h1, rec = _tc_layer1(p, x, W1l.T, b1l.reshape(1, d), W1r.T)

  agg2 = _make_agg(d, n_pad, cpw)
  q = agg2(h1, src, dst, jnp.zeros((CH, d), jnp.float32))

  return _tc_layer2(q, rec, h1, W2l.T, b2l.reshape(1, d), W2r.T)


# speedup vs baseline: 9.6823x; 1.0210x over previous
import functools

import jax
import jax.numpy as jnp
from jax import lax
from jax.experimental import pallas as pl
from jax.experimental.pallas import tpu as pltpu
from jax.experimental.pallas import tpu_sc as plsc

NC = 2
NS = 16
NW = NC * NS
CH = 128


def _make_agg(de, n_pad, cpw):
  mesh = plsc.VectorSubcoreMesh(core_axis_name="c", subcore_axis_name="s",
                                num_cores=NC, num_subcores=NS)
  rows_per_tile = n_pad // NS
  n0 = rows_per_tile // CH

  @functools.partial(
      pl.kernel,
      out_type=jax.ShapeDtypeStruct((NC, n_pad, de), jnp.float32),
      mesh=mesh,
      scratch_types=[
          pltpu.VMEM((2, CH), jnp.int32),
          pltpu.VMEM((2, CH), jnp.int32),
          pltpu.VMEM((CH, de), jnp.float32),
          pltpu.VMEM((CH, de), jnp.float32),
          pltpu.VMEM_SHARED((n_pad, de), jnp.float32),
          pltpu.SemaphoreType.DMA,
          pltpu.SemaphoreType.DMA,
          pltpu.SemaphoreType.DMA,
          pltpu.SemaphoreType.DMA,
      ],
      compiler_params=pltpu.CompilerParams(use_tc_tiling_on_sc=False),
  )
  def agg(x_hbm, edges_hbm, z_hbm, out_hbm,
          sidx, didx, rows0, rows1, acc, gsem0, gsem1, ssem0, ssem1):
    c = lax.axis_index("c")
    s = lax.axis_index("s")
    wid = c * NS + s
    rows = (rows0, rows1)
    gsems = (gsem0, gsem1)
    ssems = (ssem0, ssem1)

    pltpu.sync_copy(z_hbm, rows0)
    for k in range(n0):
      pltpu.sync_copy(rows0, acc.at[pl.ds(s * rows_per_tile + k * CH, CH)])
    plsc.subcore_barrier()

    base = wid * cpw * CH

    def load_idx(g, b):
      off = base + g * CH
      pltpu.sync_copy(edges_hbm.at[0, pl.ds(off, CH)], sidx.at[b])
      pltpu.sync_copy(edges_hbm.at[1, pl.ds(off, CH)], didx.at[b])

    def start_gather(b):
      pltpu.async_copy(x_hbm.at[sidx.at[b]], rows[b], gsems[b])

    def wait_gather(b):
      pltpu.make_async_copy(x_hbm.at[sidx.at[b]], rows[b], gsems[b]).wait()

    def start_scatter(b):
      pltpu.async_copy(rows[b], acc.at[didx.at[b]], ssems[b], add=True)

    def wait_scatter(b):
      pltpu.make_async_copy(rows[b], acc.at[didx.at[b]], ssems[b]).wait()

    load_idx(0, 0)
    start_gather(0)
    load_idx(1, 1)
    wait_gather(0)
    start_gather(1)
    start_scatter(0)

    wait_scatter(0)
    load_idx(2, 0)
    wait_gather(1)
    start_gather(0)
    start_scatter(1)

    @pl.loop(2, cpw - 1, step=2)
    def _pipe(g):
      for b in range(2):
        wait_scatter(1 - b)
        load_idx(g + b + 1, 1 - b)
        wait_gather(b)
        start_gather(1 - b)
        start_scatter(b)

    wait_scatter(1)
    wait_gather(0)
    start_scatter(0)
    wait_scatter(0)
    plsc.subcore_barrier()

    for k in range(n0):
      r0 = s * rows_per_tile + k * CH
      b = k % 2
      pltpu.sync_copy(acc.at[pl.ds(r0, CH)], rows[b])
      pltpu.sync_copy(rows[b], out_hbm.at[c, pl.ds(r0, CH)])

  return agg


def _tc_layer1(p, x, wlT, bl, wrT):
  n, d = x.shape
  dp = p.shape[2]
  bn = 1000 if n % 1000 == 0 else 8
  grid = (n // bn,)

  def body(p_r, x_r, wl_r, bl_r, wr_r, h_r, rec_r):
    a = p_r[0] + p_r[1]
    rec = 1.0 / jnp.maximum(a[:, d:d + 1], 1.0)
    mean = a[:, :d] * rec
    h = jnp.dot(mean, wl_r[...], preferred_element_type=jnp.float32)
    h = h + bl_r[...]
    h = h + jnp.dot(x_r[...], wr_r[...], preferred_element_type=jnp.float32)
    h_r[...] = jnp.maximum(h, 0.0)
    rec_r[...] = rec

  return pl.pallas_call(
      body,
      grid=grid,
      in_specs=[
          pl.BlockSpec((2, bn, dp), lambda i: (0, i, 0)),
          pl.BlockSpec((bn, d), lambda i: (i, 0)),
          pl.BlockSpec((d, d), lambda i: (0, 0)),
          pl.BlockSpec((1, d), lambda i: (0, 0)),
          pl.BlockSpec((d, d), lambda i: (0, 0)),
      ],
      out_specs=[
          pl.BlockSpec((bn, d), lambda i: (i, 0)),
          pl.BlockSpec((bn, 1), lambda i: (i, 0)),
      ],
      out_shape=[
          jax.ShapeDtypeStruct((n, d), jnp.float32),
          jax.ShapeDtypeStruct((n, 1), jnp.float32),
      ],
  )(p, x, wlT, bl, wrT)


def _tc_layer2(q, rec, h1, wlT, bl, wrT):
  n, d = h1.shape
  bn = 1000 if n % 1000 == 0 else 8
  grid = (n // bn,)

  def body(q_r, rec_r, h1_r, wl_r, bl_r, wr_r, o_r):
    mean = (q_r[0] + q_r[1]) * rec_r[...]
    h = jnp.dot(mean, wl_r[...], preferred_element_type=jnp.float32)
    h = h + bl_r[...]
    h = h + jnp.dot(h1_r[...], wr_r[...], preferred_element_type=jnp.float32)
    h = h - jnp.max(h, axis=1, keepdims=True)
    o_r[...] = h - jnp.log(jnp.sum(jnp.exp(h), axis=1, keepdims=True))

  return pl.pallas_call(
      body,
      grid=grid,
      in_specs=[
          pl.BlockSpec((2, bn, d), lambda i: (0, i, 0)),
          pl.BlockSpec((bn, 1), lambda i: (i, 0)),
          pl.BlockSpec((bn, d), lambda i: (i, 0)),
          pl.BlockSpec((d, d), lambda i: (0, 0)),
          pl.BlockSpec((1, d), lambda i: (0, 0)),
          pl.BlockSpec((d, d), lambda i: (0, 0)),
      ],
      out_specs=pl.BlockSpec((bn, d), lambda i: (i, 0)),
      out_shape=jax.ShapeDtypeStruct((n, d), jnp.float32),
  )(q, rec, h1, wlT, bl, wrT)


def kernel(x, edge_index, W1l, b1l, W1r, W2l, b2l, W2r):
  n, d = x.shape
  e = edge_index.shape[1]
  de = d + 16
  n_pad = -(-(n + 1) // (NS * CH)) * (NS * CH)
  cpw = -(-e // (NW * CH))
  if cpw % 2 == 0:
    cpw += 1
  e_pad = NW * CH * cpw

  pad = e_pad - e
  edges = edge_index
  if pad:
    pad_idx = jnp.arange(pad, dtype=jnp.int32)
    pad_blk = jnp.stack([pad_idx % n, n + pad_idx % (n_pad - n)])
    edges = jnp.concatenate([edge_index, pad_blk], axis=1)

  x_ext = jnp.concatenate(
      [x, jnp.ones((n, 1), jnp.float32), jnp.zeros((n, 15), jnp.float32)],
      axis=1)

  agg1 = _make_agg(de, n_pad, cpw)
  p = agg1(x_ext, edges, jnp.zeros((CH, de), jnp.float32))
  h1, rec = _tc_layer1(p, x, W1l.T, b1l.reshape(1, d), W1r.T)

  agg2 = _make_agg(d, n_pad, cpw)
  q = agg2(h1, edges, jnp.zeros((CH, d), jnp.float32))

  return _tc_layer2(q, rec, h1, W2l.T, b2l.reshape(1, d), W2r.T)


# speedup vs baseline: 10.0552x; 1.0385x over previous
import functools

import jax
import jax.numpy as jnp
from jax import lax
from jax.experimental import pallas as pl
from jax.experimental.pallas import tpu as pltpu
from jax.experimental.pallas import tpu_sc as plsc

NC = 2
NS = 16
NW = NC * NS
CH = 128


def _make_agg(de, n_pad, cpw):
  mesh = plsc.VectorSubcoreMesh(core_axis_name="c", subcore_axis_name="s",
                                num_cores=NC, num_subcores=NS)
  rows_per_tile = n_pad // NS
  n0 = rows_per_tile // CH

  @functools.partial(
      pl.kernel,
      out_type=jax.ShapeDtypeStruct((NC, n_pad, de), jnp.float32),
      mesh=mesh,
      scratch_types=[
          pltpu.VMEM((2, 2, CH), jnp.int32),
          pltpu.VMEM((2, 2, CH), jnp.int32),
          pltpu.VMEM((CH, de), jnp.float32),
          pltpu.VMEM((CH, de), jnp.float32),
          pltpu.VMEM_SHARED((n_pad, de), jnp.float32),
          pltpu.SemaphoreType.DMA,
          pltpu.SemaphoreType.DMA,
          pltpu.SemaphoreType.DMA,
          pltpu.SemaphoreType.DMA,
      ],
      compiler_params=pltpu.CompilerParams(use_tc_tiling_on_sc=False),
  )
  def agg(x_hbm, edges_hbm, z_hbm, out_hbm,
          sidx, didx, rows0, rows1, acc, gsem0, gsem1, ssem0, ssem1):
    c = lax.axis_index("c")
    s = lax.axis_index("s")
    wid = c * NS + s
    rows = (rows0, rows1)
    gsems = (gsem0, gsem1)
    ssems = (ssem0, ssem1)

    pltpu.sync_copy(z_hbm, rows0)
    for k in range(n0):
      pltpu.sync_copy(rows0, acc.at[pl.ds(s * rows_per_tile + k * CH, CH)])
    plsc.subcore_barrier()

    bc = wid * cpw

    def load_pair(c_next, j):
      pltpu.sync_copy(edges_hbm.at[0, pl.ds(bc + c_next, 2)], sidx.at[j])
      pltpu.sync_copy(edges_hbm.at[1, pl.ds(bc + c_next, 2)], didx.at[j])

    def start_gather(j, b):
      pltpu.async_copy(x_hbm.at[sidx.at[j, b]], rows[b], gsems[b])

    def wait_gather(j, b):
      pltpu.make_async_copy(x_hbm.at[sidx.at[j, b]], rows[b], gsems[b]).wait()

    def start_scatter(j, b):
      pltpu.async_copy(rows[b], acc.at[didx.at[j, b]], ssems[b], add=True)

    def wait_scatter(j, b):
      pltpu.make_async_copy(rows[b], acc.at[didx.at[j, b]], ssems[b]).wait()

    load_pair(0, 0)
    start_gather(0, 0)
    wait_gather(0, 0)
    start_gather(0, 1)
    start_scatter(0, 0)
    wait_scatter(0, 0)
    load_pair(2, 1)
    wait_gather(0, 1)
    start_gather(1, 0)
    start_scatter(0, 1)

    @pl.loop(2, cpw - 1, step=4)
    def _pipe(g):
      for i, (b, j) in enumerate(((0, 1), (1, 1), (0, 0), (1, 0))):
        wait_scatter(j if b == 1 else 1 - j, 1 - b)
        if b == 1:
          load_pair(g + i + 1, 1 - j)
        wait_gather(j, b)
        start_gather(j if b == 0 else 1 - j, 1 - b)
        start_scatter(j, b)

    wait_scatter(0, 1)
    wait_gather(1, 0)
    start_scatter(1, 0)
    wait_scatter(1, 0)
    plsc.subcore_barrier()

    for k in range(n0):
      r0 = s * rows_per_tile + k * CH
      b = k % 2
      pltpu.sync_copy(acc.at[pl.ds(r0, CH)], rows[b])
      pltpu.sync_copy(rows[b], out_hbm.at[c, pl.ds(r0, CH)])

  return agg


def _tc_layer1(p, x, wlT, bl, wrT):
  n, d = x.shape
  dp = p.shape[2]
  bn = 1000 if n % 1000 == 0 else 8
  grid = (n // bn,)

  def body(p_r, x_r, wl_r, bl_r, wr_r, h_r, rec_r):
    a = p_r[0] + p_r[1]
    rec = 1.0 / jnp.maximum(a[:, d:d + 1], 1.0)
    mean = a[:, :d] * rec
    h = jnp.dot(mean, wl_r[...], preferred_element_type=jnp.float32)
    h = h + bl_r[...]
    h = h + jnp.dot(x_r[...], wr_r[...], preferred_element_type=jnp.float32)
    h_r[...] = jnp.maximum(h, 0.0)
    rec_r[...] = rec

  return pl.pallas_call(
      body,
      grid=grid,
      in_specs=[
          pl.BlockSpec((2, bn, dp), lambda i: (0, i, 0)),
          pl.BlockSpec((bn, d), lambda i: (i, 0)),
          pl.BlockSpec((d, d), lambda i: (0, 0)),
          pl.BlockSpec((1, d), lambda i: (0, 0)),
          pl.BlockSpec((d, d), lambda i: (0, 0)),
      ],
      out_specs=[
          pl.BlockSpec((bn, d), lambda i: (i, 0)),
          pl.BlockSpec((bn, 1), lambda i: (i, 0)),
      ],
      out_shape=[
          jax.ShapeDtypeStruct((n, d), jnp.float32),
          jax.ShapeDtypeStruct((n, 1), jnp.float32),
      ],
  )(p, x, wlT, bl, wrT)


def _tc_layer2(q, rec, h1, wlT, bl, wrT):
  n, d = h1.shape
  bn = 1000 if n % 1000 == 0 else 8
  grid = (n // bn,)

  def body(q_r, rec_r, h1_r, wl_r, bl_r, wr_r, o_r):
    mean = (q_r[0] + q_r[1]) * rec_r[...]
    h = jnp.dot(mean, wl_r[...], preferred_element_type=jnp.float32)
    h = h + bl_r[...]
    h = h + jnp.dot(h1_r[...], wr_r[...], preferred_element_type=jnp.float32)
    h = h - jnp.max(h, axis=1, keepdims=True)
    o_r[...] = h - jnp.log(jnp.sum(jnp.exp(h), axis=1, keepdims=True))

  return pl.pallas_call(
      body,
      grid=grid,
      in_specs=[
          pl.BlockSpec((2, bn, d), lambda i: (0, i, 0)),
          pl.BlockSpec((bn, 1), lambda i: (i, 0)),
          pl.BlockSpec((bn, d), lambda i: (i, 0)),
          pl.BlockSpec((d, d), lambda i: (0, 0)),
          pl.BlockSpec((1, d), lambda i: (0, 0)),
          pl.BlockSpec((d, d), lambda i: (0, 0)),
      ],
      out_specs=pl.BlockSpec((bn, d), lambda i: (i, 0)),
      out_shape=jax.ShapeDtypeStruct((n, d), jnp.float32),
  )(q, rec, h1, wlT, bl, wrT)


def kernel(x, edge_index, W1l, b1l, W1r, W2l, b2l, W2r):
  n, d = x.shape
  e = edge_index.shape[1]
  de = d + 16
  n_pad = -(-(n + 1) // (NS * CH)) * (NS * CH)
  cpw = -(-e // (NW * CH))
  while cpw % 4 != 3:
    cpw += 1
  e_pad = NW * CH * cpw

  pad = e_pad - e
  edges = edge_index
  if pad:
    pad_idx = jnp.arange(pad, dtype=jnp.int32)
    pad_blk = jnp.stack([pad_idx % n, n + pad_idx % (n_pad - n)])
    edges = jnp.concatenate([edge_index, pad_blk], axis=1)
  edges = edges.reshape(2, e_pad // CH, CH)

  x_ext = jnp.concatenate(
      [x, jnp.ones((n, 1), jnp.float32), jnp.zeros((n, 15), jnp.float32)],
      axis=1)

  agg1 = _make_agg(de, n_pad, cpw)
  p = agg1(x_ext, edges, jnp.zeros((CH, de), jnp.float32))
  h1, rec = _tc_layer1(p, x, W1l.T, b1l.reshape(1, d), W1r.T)

  agg2 = _make_agg(d, n_pad, cpw)
  q = agg2(h1, edges, jnp.zeros((CH, d), jnp.float32))

  return _tc_layer2(q, rec, h1, W2l.T, b2l.reshape(1, d), W2r.T)


# speedup vs baseline: 10.1597x; 1.0104x over previous
import functools

import jax
import jax.numpy as jnp
from jax import lax
from jax.experimental import pallas as pl
from jax.experimental.pallas import tpu as pltpu
from jax.experimental.pallas import tpu_sc as plsc

NC = 2
NS = 16
NW = NC * NS
CH = 128


def _make_agg(de, n_pad, cpw):
  mesh = plsc.VectorSubcoreMesh(core_axis_name="c", subcore_axis_name="s",
                                num_cores=NC, num_subcores=NS)
  rows_per_tile = n_pad // NS
  n0 = rows_per_tile // CH

  @functools.partial(
      pl.kernel,
      out_type=jax.ShapeDtypeStruct((NC, n_pad, de), jnp.float32),
      mesh=mesh,
      scratch_types=[
          pltpu.VMEM((2, 2, CH), jnp.int32),
          pltpu.VMEM((2, 2, CH), jnp.int32),
          pltpu.VMEM((CH, de), jnp.float32),
          pltpu.VMEM((CH, de), jnp.float32),
          pltpu.VMEM_SHARED((n_pad, de), jnp.float32),
          pltpu.SemaphoreType.DMA,
          pltpu.SemaphoreType.DMA,
          pltpu.SemaphoreType.DMA,
          pltpu.SemaphoreType.DMA,
      ],
      compiler_params=pltpu.CompilerParams(use_tc_tiling_on_sc=False),
  )
  def agg(x_hbm, edges_hbm, z_hbm, out_hbm,
          sidx, didx, rows0, rows1, acc, gsem0, gsem1, ssem0, ssem1):
    c = lax.axis_index("c")
    s = lax.axis_index("s")
    wid = c * NS + s
    rows = (rows0, rows1)
    gsems = (gsem0, gsem1)
    ssems = (ssem0, ssem1)

    pltpu.sync_copy(z_hbm, rows0)
    for k in range(n0):
      pltpu.async_copy(rows0, acc.at[pl.ds(s * rows_per_tile + k * CH, CH)],
                       gsem0)
    for k in range(n0):
      pltpu.make_async_copy(
          rows0, acc.at[pl.ds(s * rows_per_tile + k * CH, CH)], gsem0).wait()
    plsc.subcore_barrier()

    bc = wid * cpw

    def load_pair(c_next, j):
      pltpu.sync_copy(edges_hbm.at[0, pl.ds(bc + c_next, 2)], sidx.at[j])
      pltpu.sync_copy(edges_hbm.at[1, pl.ds(bc + c_next, 2)], didx.at[j])

    def start_gather(j, b):
      pltpu.async_copy(x_hbm.at[sidx.at[j, b]], rows[b], gsems[b])

    def wait_gather(j, b):
      pltpu.make_async_copy(x_hbm.at[sidx.at[j, b]], rows[b], gsems[b]).wait()

    def start_scatter(j, b):
      pltpu.async_copy(rows[b], acc.at[didx.at[j, b]], ssems[b], add=True)

    def wait_scatter(j, b):
      pltpu.make_async_copy(rows[b], acc.at[didx.at[j, b]], ssems[b]).wait()

    load_pair(0, 0)
    start_gather(0, 0)
    wait_gather(0, 0)
    start_gather(0, 1)
    start_scatter(0, 0)
    wait_scatter(0, 0)
    load_pair(2, 1)
    wait_gather(0, 1)
    start_gather(1, 0)
    start_scatter(0, 1)

    @pl.loop(2, cpw - 1, step=4)
    def _pipe(g):
      for i, (b, j) in enumerate(((0, 1), (1, 1), (0, 0), (1, 0))):
        wait_scatter(j if b == 1 else 1 - j, 1 - b)
        if b == 1:
          load_pair(g + i + 1, 1 - j)
        wait_gather(j, b)
        start_gather(j if b == 0 else 1 - j, 1 - b)
        start_scatter(j, b)

    wait_scatter(0, 1)
    wait_gather(1, 0)
    start_scatter(1, 0)
    wait_scatter(1, 0)
    plsc.subcore_barrier()

    def drain_r0(k):
      return s * rows_per_tile + k * CH

    for k in range(n0):
      b = k % 2
      if k >= 2:
        pltpu.make_async_copy(
            rows[b], out_hbm.at[c, pl.ds(drain_r0(k - 2), CH)],
            ssems[b]).wait()
      pltpu.sync_copy(acc.at[pl.ds(drain_r0(k), CH)], rows[b])
      pltpu.async_copy(rows[b], out_hbm.at[c, pl.ds(drain_r0(k), CH)],
                       ssems[b])
    for k in range(max(n0 - 2, 0), n0):
      pltpu.make_async_copy(
          rows[k % 2], out_hbm.at[c, pl.ds(drain_r0(k), CH)],
          ssems[k % 2]).wait()

  return agg


def _tc_layer1(p, x, wlT, bl, wrT):
  n, d = x.shape
  dp = p.shape[2]
  bn = 1000 if n % 1000 == 0 else 8
  grid = (n // bn,)

  def body(p_r, x_r, wl_r, bl_r, wr_r, h_r, rec_r):
    a = p_r[0] + p_r[1]
    rec = 1.0 / jnp.maximum(a[:, d:d + 1], 1.0)
    mean = a[:, :d] * rec
    h = jnp.dot(mean, wl_r[...], preferred_element_type=jnp.float32)
    h = h + bl_r[...]
    h = h + jnp.dot(x_r[...], wr_r[...], preferred_element_type=jnp.float32)
    h_r[...] = jnp.maximum(h, 0.0)
    rec_r[...] = rec

  return pl.pallas_call(
      body,
      grid=grid,
      in_specs=[
          pl.BlockSpec((2, bn, dp), lambda i: (0, i, 0)),
          pl.BlockSpec((bn, d), lambda i: (i, 0)),
          pl.BlockSpec((d, d), lambda i: (0, 0)),
          pl.BlockSpec((1, d), lambda i: (0, 0)),
          pl.BlockSpec((d, d), lambda i: (0, 0)),
      ],
      out_specs=[
          pl.BlockSpec((bn, d), lambda i: (i, 0)),
          pl.BlockSpec((bn, 1), lambda i: (i, 0)),
      ],
      out_shape=[
          jax.ShapeDtypeStruct((n, d), jnp.float32),
          jax.ShapeDtypeStruct((n, 1), jnp.float32),
      ],
  )(p, x, wlT, bl, wrT)


def _tc_layer2(q, rec, h1, wlT, bl, wrT):
  n, d = h1.shape
  bn = 1000 if n % 1000 == 0 else 8
  grid = (n // bn,)

  def body(q_r, rec_r, h1_r, wl_r, bl_r, wr_r, o_r):
    mean = (q_r[0] + q_r[1]) * rec_r[...]
    h = jnp.dot(mean, wl_r[...], preferred_element_type=jnp.float32)
    h = h + bl_r[...]
    h = h + jnp.dot(h1_r[...], wr_r[...], preferred_element_type=jnp.float32)
    h = h - jnp.max(h, axis=1, keepdims=True)
    o_r[...] = h - jnp.log(jnp.sum(jnp.exp(h), axis=1, keepdims=True))

  return pl.pallas_call(
      body,
      grid=grid,
      in_specs=[
          pl.BlockSpec((2, bn, d), lambda i: (0, i, 0)),
          pl.BlockSpec((bn, 1), lambda i: (i, 0)),
          pl.BlockSpec((bn, d), lambda i: (i, 0)),
          pl.BlockSpec((d, d), lambda i: (0, 0)),
          pl.BlockSpec((1, d), lambda i: (0, 0)),
          pl.BlockSpec((d, d), lambda i: (0, 0)),
      ],
      out_specs=pl.BlockSpec((bn, d), lambda i: (i, 0)),
      out_shape=jax.ShapeDtypeStruct((n, d), jnp.float32),
  )(q, rec, h1, wlT, bl, wrT)


def kernel(x, edge_index, W1l, b1l, W1r, W2l, b2l, W2r):
  n, d = x.shape
  e = edge_index.shape[1]
  de = d + 16
  n_pad = -(-(n + 1) // (NS * CH)) * (NS * CH)
  cpw = -(-e // (NW * CH))
  while cpw % 4 != 3:
    cpw += 1
  e_pad = NW * CH * cpw

  pad = e_pad - e
  edges = edge_index
  if pad:
    pad_idx = jnp.arange(pad, dtype=jnp.int32)
    pad_blk = jnp.stack([pad_idx % n, n + pad_idx % (n_pad - n)])
    edges = jnp.concatenate([edge_index, pad_blk], axis=1)
  edges = edges.reshape(2, e_pad // CH, CH)

  x_ext = jnp.concatenate(
      [x, jnp.ones((n, 1), jnp.float32), jnp.zeros((n, 15), jnp.float32)],
      axis=1)

  agg1 = _make_agg(de, n_pad, cpw)
  p = agg1(x_ext, edges, jnp.zeros((CH, de), jnp.float32))
  h1, rec = _tc_layer1(p, x, W1l.T, b1l.reshape(1, d), W1r.T)

  agg2 = _make_agg(d, n_pad, cpw)
  q = agg2(h1, edges, jnp.zeros((CH, d), jnp.float32))

  return _tc_layer2(q, rec, h1, W2l.T, b2l.reshape(1, d), W2r.T)


# speedup vs baseline: 10.7958x; 1.0626x over previous
import functools

import jax
import jax.numpy as jnp
from jax import lax
from jax.experimental import pallas as pl
from jax.experimental.pallas import tpu as pltpu
from jax.experimental.pallas import tpu_sc as plsc

NC = 2
NS = 16
NW = NC * NS
CH = 128


def _make_agg(de, n_pad, cpw):
  mesh = plsc.VectorSubcoreMesh(core_axis_name="c", subcore_axis_name="s",
                                num_cores=NC, num_subcores=NS)
  rows_per_tile = n_pad // NS
  n0 = rows_per_tile // CH

  @functools.partial(
      pl.kernel,
      out_type=jax.ShapeDtypeStruct((NC, n_pad, de), jnp.float32),
      mesh=mesh,
      scratch_types=[
          pltpu.VMEM((2, 2, CH), jnp.int32),
          pltpu.VMEM((2, 2, CH), jnp.int32),
          pltpu.VMEM((CH, de), jnp.float32),
          pltpu.VMEM((CH, de), jnp.float32),
          pltpu.VMEM_SHARED((n_pad, de), jnp.float32),
          pltpu.SemaphoreType.DMA,
          pltpu.SemaphoreType.DMA,
          pltpu.SemaphoreType.DMA,
          pltpu.SemaphoreType.DMA,
          pltpu.SemaphoreType.DMA,
          pltpu.SemaphoreType.DMA,
      ],
      compiler_params=pltpu.CompilerParams(use_tc_tiling_on_sc=False),
  )
  def agg(x_hbm, edges_hbm, z_hbm, out_hbm,
          sidx, didx, rows0, rows1, acc,
          gsem0, gsem1, ssem0, ssem1, isem0, isem1):
    c = lax.axis_index("c")
    s = lax.axis_index("s")
    wid = c * NS + s
    rows = (rows0, rows1)
    gsems = (gsem0, gsem1)
    ssems = (ssem0, ssem1)
    isems = (isem0, isem1)

    pltpu.sync_copy(z_hbm, rows0)
    for k in range(n0):
      pltpu.async_copy(rows0, acc.at[pl.ds(s * rows_per_tile + k * CH, CH)],
                       gsem0)
    for k in range(n0):
      pltpu.make_async_copy(
          rows0, acc.at[pl.ds(s * rows_per_tile + k * CH, CH)], gsem0).wait()
    plsc.subcore_barrier()

    bc = wid * cpw

    def start_load_pair(c_next, j):
      pltpu.async_copy(edges_hbm.at[0, pl.ds(bc + c_next, 2)], sidx.at[j],
                       isems[j])
      pltpu.async_copy(edges_hbm.at[1, pl.ds(bc + c_next, 2)], didx.at[j],
                       isems[j])

    def wait_load_pair(c_next, j):
      pltpu.make_async_copy(edges_hbm.at[0, pl.ds(bc + c_next, 2)],
                            sidx.at[j], isems[j]).wait()
      pltpu.make_async_copy(edges_hbm.at[1, pl.ds(bc + c_next, 2)],
                            didx.at[j], isems[j]).wait()

    def start_gather(j, b):
      pltpu.async_copy(x_hbm.at[sidx.at[j, b]], rows[b], gsems[b])

    def wait_gather(j, b):
      pltpu.make_async_copy(x_hbm.at[sidx.at[j, b]], rows[b], gsems[b]).wait()

    def start_scatter(j, b):
      pltpu.async_copy(rows[b], acc.at[didx.at[j, b]], ssems[b], add=True)

    def wait_scatter(j, b):
      pltpu.make_async_copy(rows[b], acc.at[didx.at[j, b]], ssems[b]).wait()

    start_load_pair(0, 0)
    wait_load_pair(0, 0)
    start_gather(0, 0)
    start_load_pair(2, 1)
    wait_gather(0, 0)
    start_gather(0, 1)
    start_scatter(0, 0)
    wait_scatter(0, 0)
    wait_gather(0, 1)
    wait_load_pair(2, 1)
    start_gather(1, 0)
    start_scatter(0, 1)

    @pl.loop(2, cpw - 1, step=4)
    def _pipe(g):
      for i, (b, j) in enumerate(((0, 1), (1, 1), (0, 0), (1, 0))):
        c = g + i
        wait_scatter(j if b == 1 else 1 - j, 1 - b)
        if b == 0:
          start_load_pair(c + 2, 1 - j)
        wait_gather(j, b)
        if b == 1:
          wait_load_pair(c + 1, 1 - j)
        start_gather(j if b == 0 else 1 - j, 1 - b)
        start_scatter(j, b)

    wait_scatter(0, 1)
    wait_gather(1, 0)
    start_scatter(1, 0)
    wait_scatter(1, 0)
    plsc.subcore_barrier()

    def drain_r0(k):
      return s * rows_per_tile + k * CH

    for k in range(n0):
      b = k % 2
      if k >= 2:
        pltpu.make_async_copy(
            rows[b], out_hbm.at[c, pl.ds(drain_r0(k - 2), CH)],
            ssems[b]).wait()
      pltpu.sync_copy(acc.at[pl.ds(drain_r0(k), CH)], rows[b])
      pltpu.async_copy(rows[b], out_hbm.at[c, pl.ds(drain_r0(k), CH)],
                       ssems[b])
    for k in range(max(n0 - 2, 0), n0):
      pltpu.make_async_copy(
          rows[k % 2], out_hbm.at[c, pl.ds(drain_r0(k), CH)],
          ssems[k % 2]).wait()

  return agg


def _tc_layer1(p, x, wlT, bl, wrT):
  n, d = x.shape
  dp = p.shape[2]
  bn = 1000 if n % 1000 == 0 else 8
  grid = (n // bn,)

  def body(p_r, x_r, wl_r, bl_r, wr_r, h_r, rec_r):
    a = p_r[0] + p_r[1]
    rec = 1.0 / jnp.maximum(a[:, d:d + 1], 1.0)
    mean = a[:, :d] * rec
    h = jnp.dot(mean, wl_r[...], preferred_element_type=jnp.float32)
    h = h + bl_r[...]
    h = h + jnp.dot(x_r[...], wr_r[...], preferred_element_type=jnp.float32)
    h_r[...] = jnp.maximum(h, 0.0)
    rec_r[...] = rec

  return pl.pallas_call(
      body,
      grid=grid,
      in_specs=[
          pl.BlockSpec((2, bn, dp), lambda i: (0, i, 0)),
          pl.BlockSpec((bn, d), lambda i: (i, 0)),
          pl.BlockSpec((d, d), lambda i: (0, 0)),
          pl.BlockSpec((1, d), lambda i: (0, 0)),
          pl.BlockSpec((d, d), lambda i: (0, 0)),
      ],
      out_specs=[
          pl.BlockSpec((bn, d), lambda i: (i, 0)),
          pl.BlockSpec((bn, 1), lambda i: (i, 0)),
      ],
      out_shape=[
          jax.ShapeDtypeStruct((n, d), jnp.float32),
          jax.ShapeDtypeStruct((n, 1), jnp.float32),
      ],
  )(p, x, wlT, bl, wrT)


def _tc_layer2(q, rec, h1, wlT, bl, wrT):
  n, d = h1.shape
  bn = 1000 if n % 1000 == 0 else 8
  grid = (n // bn,)

  def body(q_r, rec_r, h1_r, wl_r, bl_r, wr_r, o_r):
    mean = (q_r[0] + q_r[1]) * rec_r[...]
    h = jnp.dot(mean, wl_r[...], preferred_element_type=jnp.float32)
    h = h + bl_r[...]
    h = h + jnp.dot(h1_r[...], wr_r[...], preferred_element_type=jnp.float32)
    h = h - jnp.max(h, axis=1, keepdims=True)
    o_r[...] = h - jnp.log(jnp.sum(jnp.exp(h), axis=1, keepdims=True))

  return pl.pallas_call(
      body,
      grid=grid,
      in_specs=[
          pl.BlockSpec((2, bn, d), lambda i: (0, i, 0)),
          pl.BlockSpec((bn, 1), lambda i: (i, 0)),
          pl.BlockSpec((bn, d), lambda i: (i, 0)),
          pl.BlockSpec((d, d), lambda i: (0, 0)),
          pl.BlockSpec((1, d), lambda i: (0, 0)),
          pl.BlockSpec((d, d), lambda i: (0, 0)),
      ],
      out_specs=pl.BlockSpec((bn, d), lambda i: (i, 0)),
      out_shape=jax.ShapeDtypeStruct((n, d), jnp.float32),
  )(q, rec, h1, wlT, bl, wrT)


def kernel(x, edge_index, W1l, b1l, W1r, W2l, b2l, W2r):
  n, d = x.shape
  e = edge_index.shape[1]
  de = d + 16
  n_pad = -(-(n + 1) // (NS * CH)) * (NS * CH)
  cpw = -(-e // (NW * CH))
  while cpw % 4 != 3:
    cpw += 1
  e_pad = NW * CH * cpw

  pad = e_pad + CH - e
  edges = edge_index
  if pad:
    pad_idx = jnp.arange(pad, dtype=jnp.int32)
    pad_blk = jnp.stack([pad_idx % n, n + pad_idx % (n_pad - n)])
    edges = jnp.concatenate([edge_index, pad_blk], axis=1)
  edges = edges.reshape(2, e_pad // CH + 1, CH)

  x_ext = jnp.concatenate(
      [x, jnp.ones((n, 1), jnp.float32), jnp.zeros((n, 15), jnp.float32)],
      axis=1)

  agg1 = _make_agg(de, n_pad, cpw)
  p = agg1(x_ext, edges, jnp.zeros((CH, de), jnp.float32))
  h1, rec = _tc_layer1(p, x, W1l.T, b1l.reshape(1, d), W1r.T)

  agg2 = _make_agg(d, n_pad, cpw)
  q = agg2(h1, edges, jnp.zeros((CH, d), jnp.float32))

  return _tc_layer2(q, rec, h1, W2l.T, b2l.reshape(1, d), W2r.T)
